# Initial kernel scaffold; baseline (speedup 1.0000x reference)
import jax
import jax.numpy as jnp
from jax.experimental import pallas as pl


def kernel(x, adj, fc_W, fc_b, conv_W, dyn_W, dyn_b, ln1_g, ln1_b, ln2_g, ln2_b, out_W, out_b):
    raise NotImplementedError("write your pallas kernel here")


# speedup vs baseline: 1.0540x; 1.0540x over previous
import math

import jax
import jax.numpy as jnp
from jax.experimental import pallas as pl
from jax.experimental.pallas import tpu as pltpu

N = 10000
NFEAT = 128
NHID = 128
NCLASS = 40
LAMDA = 0.5
ALPHA = 0.1

BM = 200
BM0 = 2000


def _h0_body(x_ref, w_ref, b_ref, o_ref):
    o_ref[...] = jax.nn.relu(
        jnp.dot(x_ref[...], w_ref[...], preferred_element_type=jnp.float32)
        + b_ref[...])


def _layer1_body(adj_ref, h0_ref, m_ref, b_ref, o_ref):
    i = pl.program_id(0)
    hi = jnp.dot(adj_ref[...], h0_ref[...], preferred_element_type=jnp.float32)
    h0_rows = h0_ref[pl.ds(i * BM, BM), :]
    o_ref[...] = jax.nn.relu(
        jnp.dot(hi, m_ref[...], preferred_element_type=jnp.float32)
        + jnp.dot(h0_rows, b_ref[...], preferred_element_type=jnp.float32))


def _ln(z, g, b):
    m = jnp.mean(z, axis=-1, keepdims=True)
    c = z - m
    v = jnp.mean(c * c, axis=-1, keepdims=True)
    return g * c * jax.lax.rsqrt(v + 1e-6) + b


def _layer2_tail_body(adj_ref, h1_ref, h0_ref, m_ref, b_ref,
                      wd0_ref, bd0_ref, wd1_ref, bd1_ref,
                      ln1g_ref, ln1b_ref, ln2g_ref, ln2b_ref,
                      ow_ref, ob_ref, logp_ref, cross_ref):
    i = pl.program_id(0)
    hi = jnp.dot(adj_ref[...], h1_ref[...], preferred_element_type=jnp.float32)
    h1_rows = h1_ref[pl.ds(i * BM, BM), :]
    h0 = h0_ref[...]
    h2 = jax.nn.relu(
        jnp.dot(hi, m_ref[...], preferred_element_type=jnp.float32)
        + jnp.dot(h0, b_ref[...], preferred_element_type=jnp.float32)
        + h1_rows)
    dy = jax.nn.relu(
        jnp.dot(h0, wd0_ref[...], preferred_element_type=jnp.float32)
        + bd0_ref[...])
    dy = jax.nn.relu(
        jnp.dot(dy, wd1_ref[...], preferred_element_type=jnp.float32)
        + bd1_ref[...]) + 0.1 * h0
    cross = (_ln(h2, ln1g_ref[...], ln1b_ref[...])
             + _ln(dy, ln2g_ref[...], ln2b_ref[...]))
    cross_ref[...] = cross
    logits = jnp.dot(cross, ow_ref[...], preferred_element_type=jnp.float32) \
        + ob_ref[...]
    mx = jnp.max(logits, axis=-1, keepdims=True)
    s = logits - mx
    logp_ref[...] = s - jnp.log(jnp.sum(jnp.exp(s), axis=-1, keepdims=True))


def kernel(x, adj, fc_W, fc_b, conv_W, dyn_W, dyn_b,
           ln1_g, ln1_b, ln2_g, ln2_b, out_W, out_b):
    f32 = jnp.float32
    eye = jnp.eye(NHID, dtype=f32)
    th1 = math.log(LAMDA / 1.0 + 1.0)
    th2 = math.log(LAMDA / 2.0 + 1.0)
    M1 = th1 * conv_W[0, :NHID] + (1.0 - th1) * (1.0 - ALPHA) * eye
    B1 = th1 * conv_W[0, NHID:] + ((1.0 - th1) * ALPHA + 1.0) * eye
    M2 = th2 * conv_W[1, :NHID] + (1.0 - th2) * (1.0 - ALPHA) * eye
    B2 = th2 * conv_W[1, NHID:] + (1.0 - th2) * ALPHA * eye
    Wd0 = dyn_W[0, :NHID] - dyn_W[0, NHID:]
    Wd1 = dyn_W[1, :NHID] - dyn_W[1, NHID:]
    bd0 = dyn_b[0].reshape(1, NHID)
    bd1 = dyn_b[1].reshape(1, NHID)

    h0 = pl.pallas_call(
        _h0_body,
        grid=(N // BM0,),
        in_specs=[
            pl.BlockSpec((BM0, NFEAT), lambda i: (i, 0)),
            pl.BlockSpec((NFEAT, NHID), lambda i: (0, 0)),
            pl.BlockSpec((1, NHID), lambda i: (0, 0)),
        ],
        out_specs=pl.BlockSpec((BM0, NHID), lambda i: (i, 0)),
        out_shape=jax.ShapeDtypeStruct((N, NHID), f32),
        compiler_params=pltpu.CompilerParams(
            dimension_semantics=("parallel",)),
    )(x, fc_W, fc_b.reshape(1, NHID))

    h1 = pl.pallas_call(
        _layer1_body,
        grid=(N // BM,),
        in_specs=[
            pl.BlockSpec((BM, N), lambda i: (i, 0)),
            pl.BlockSpec((N, NHID), lambda i: (0, 0)),
            pl.BlockSpec((NHID, NHID), lambda i: (0, 0)),
            pl.BlockSpec((NHID, NHID), lambda i: (0, 0)),
        ],
        out_specs=pl.BlockSpec((BM, NHID), lambda i: (i, 0)),
        out_shape=jax.ShapeDtypeStruct((N, NHID), f32),
        compiler_params=pltpu.CompilerParams(
            dimension_semantics=("parallel",)),
    )(adj, h0, M1, B1)

    logp, cross = pl.pallas_call(
        _layer2_tail_body,
        grid=(N // BM,),
        in_specs=[
            pl.BlockSpec((BM, N), lambda i: (i, 0)),
            pl.BlockSpec((N, NHID), lambda i: (0, 0)),
            pl.BlockSpec((BM, NHID), lambda i: (i, 0)),
            pl.BlockSpec((NHID, NHID), lambda i: (0, 0)),
            pl.BlockSpec((NHID, NHID), lambda i: (0, 0)),
            pl.BlockSpec((NHID, NHID), lambda i: (0, 0)),
            pl.BlockSpec((1, NHID), lambda i: (0, 0)),
            pl.BlockSpec((NHID, NHID), lambda i: (0, 0)),
            pl.BlockSpec((1, NHID), lambda i: (0, 0)),
            pl.BlockSpec((1, NHID), lambda i: (0, 0)),
            pl.BlockSpec((1, NHID), lambda i: (0, 0)),
            pl.BlockSpec((1, NHID), lambda i: (0, 0)),
            pl.BlockSpec((1, NHID), lambda i: (0, 0)),
            pl.BlockSpec((NHID, NCLASS), lambda i: (0, 0)),
            pl.BlockSpec((1, NCLASS), lambda i: (0, 0)),
        ],
        out_specs=[
            pl.BlockSpec((BM, NCLASS), lambda i: (i, 0)),
            pl.BlockSpec((BM, NHID), lambda i: (i, 0)),
        ],
        out_shape=[
            jax.ShapeDtypeStruct((N, NCLASS), f32),
            jax.ShapeDtypeStruct((N, NHID), f32),
        ],
        compiler_params=pltpu.CompilerParams(
            dimension_semantics=("parallel",)),
    )(adj, h1, h0, M2, B2, Wd0, bd0, Wd1, bd1,
      ln1_g.reshape(1, NHID), ln1_b.reshape(1, NHID),
      ln2_g.reshape(1, NHID), ln2_b.reshape(1, NHID),
      out_W, out_b.reshape(1, NCLASS))

    return (logp, cross)


# speedup vs baseline: 1.1075x; 1.0507x over previous
import math

import jax
import jax.numpy as jnp
from jax.experimental import pallas as pl
from jax.experimental.pallas import tpu as pltpu

N = 10000
NFEAT = 128
NHID = 128
NCLASS = 40
LAMDA = 0.5
ALPHA = 0.1

BM = 400
BM0 = 2000


def _h0_body(x_ref, w_ref, b_ref, o_ref):
    o_ref[...] = jax.nn.relu(
        jnp.dot(x_ref[...], w_ref[...], preferred_element_type=jnp.float32)
        + b_ref[...])


def _layer1_body(adj_ref, h0_ref, m_ref, b_ref, o_ref):
    i = pl.program_id(0)
    hi = jnp.dot(adj_ref[...], h0_ref[...], preferred_element_type=jnp.float32)
    h0_rows = h0_ref[pl.ds(i * BM, BM), :]
    o_ref[...] = jax.nn.relu(
        jnp.dot(hi, m_ref[...], preferred_element_type=jnp.float32)
        + jnp.dot(h0_rows, b_ref[...], preferred_element_type=jnp.float32))


def _ln(z, g, b):
    m = jnp.mean(z, axis=-1, keepdims=True)
    c = z - m
    v = jnp.mean(c * c, axis=-1, keepdims=True)
    return g * c * jax.lax.rsqrt(v + 1e-6) + b


def _layer2_tail_body(adj_ref, h1_ref, h0_ref, m_ref, b_ref,
                      wd0_ref, bd0_ref, wd1_ref, bd1_ref,
                      ln1g_ref, ln1b_ref, ln2g_ref, ln2b_ref,
                      ow_ref, ob_ref, logp_ref, cross_ref):
    i = pl.program_id(0)
    hi = jnp.dot(adj_ref[...], h1_ref[...], preferred_element_type=jnp.float32)
    h1_rows = h1_ref[pl.ds(i * BM, BM), :]
    h0 = h0_ref[...]
    h2 = jax.nn.relu(
        jnp.dot(hi, m_ref[...], preferred_element_type=jnp.float32)
        + jnp.dot(h0, b_ref[...], preferred_element_type=jnp.float32)
        + h1_rows)
    dy = jax.nn.relu(
        jnp.dot(h0, wd0_ref[...], preferred_element_type=jnp.float32)
        + bd0_ref[...])
    dy = jax.nn.relu(
        jnp.dot(dy, wd1_ref[...], preferred_element_type=jnp.float32)
        + bd1_ref[...]) + 0.1 * h0
    cross = (_ln(h2, ln1g_ref[...], ln1b_ref[...])
             + _ln(dy, ln2g_ref[...], ln2b_ref[...]))
    cross_ref[...] = cross
    logits = jnp.dot(cross, ow_ref[...], preferred_element_type=jnp.float32) \
        + ob_ref[...]
    mx = jnp.max(logits, axis=-1, keepdims=True)
    s = logits - mx
    logp_ref[...] = s - jnp.log(jnp.sum(jnp.exp(s), axis=-1, keepdims=True))


def kernel(x, adj, fc_W, fc_b, conv_W, dyn_W, dyn_b,
           ln1_g, ln1_b, ln2_g, ln2_b, out_W, out_b):
    f32 = jnp.float32
    eye = jnp.eye(NHID, dtype=f32)
    th1 = math.log(LAMDA / 1.0 + 1.0)
    th2 = math.log(LAMDA / 2.0 + 1.0)
    M1 = th1 * conv_W[0, :NHID] + (1.0 - th1) * (1.0 - ALPHA) * eye
    B1 = th1 * conv_W[0, NHID:] + ((1.0 - th1) * ALPHA + 1.0) * eye
    M2 = th2 * conv_W[1, :NHID] + (1.0 - th2) * (1.0 - ALPHA) * eye
    B2 = th2 * conv_W[1, NHID:] + (1.0 - th2) * ALPHA * eye
    Wd0 = dyn_W[0, :NHID] - dyn_W[0, NHID:]
    Wd1 = dyn_W[1, :NHID] - dyn_W[1, NHID:]
    bd0 = dyn_b[0].reshape(1, NHID)
    bd1 = dyn_b[1].reshape(1, NHID)

    h0 = pl.pallas_call(
        _h0_body,
        grid=(N // BM0,),
        in_specs=[
            pl.BlockSpec((BM0, NFEAT), lambda i: (i, 0)),
            pl.BlockSpec((NFEAT, NHID), lambda i: (0, 0)),
            pl.BlockSpec((1, NHID), lambda i: (0, 0)),
        ],
        out_specs=pl.BlockSpec((BM0, NHID), lambda i: (i, 0)),
        out_shape=jax.ShapeDtypeStruct((N, NHID), f32),
        compiler_params=pltpu.CompilerParams(
            dimension_semantics=("parallel",)),
    )(x, fc_W, fc_b.reshape(1, NHID))

    h1 = pl.pallas_call(
        _layer1_body,
        grid=(N // BM,),
        in_specs=[
            pl.BlockSpec((BM, N), lambda i: (i, 0)),
            pl.BlockSpec((N, NHID), lambda i: (0, 0)),
            pl.BlockSpec((NHID, NHID), lambda i: (0, 0)),
            pl.BlockSpec((NHID, NHID), lambda i: (0, 0)),
        ],
        out_specs=pl.BlockSpec((BM, NHID), lambda i: (i, 0)),
        out_shape=jax.ShapeDtypeStruct((N, NHID), f32),
        compiler_params=pltpu.CompilerParams(
            dimension_semantics=("parallel",)),
    )(adj, h0, M1, B1)

    logp, cross = pl.pallas_call(
        _layer2_tail_body,
        grid=(N // BM,),
        in_specs=[
            pl.BlockSpec((BM, N), lambda i: (i, 0)),
            pl.BlockSpec((N, NHID), lambda i: (0, 0)),
            pl.BlockSpec((BM, NHID), lambda i: (i, 0)),
            pl.BlockSpec((NHID, NHID), lambda i: (0, 0)),
            pl.BlockSpec((NHID, NHID), lambda i: (0, 0)),
            pl.BlockSpec((NHID, NHID), lambda i: (0, 0)),
            pl.BlockSpec((1, NHID), lambda i: (0, 0)),
            pl.BlockSpec((NHID, NHID), lambda i: (0, 0)),
            pl.BlockSpec((1, NHID), lambda i: (0, 0)),
            pl.BlockSpec((1, NHID), lambda i: (0, 0)),
            pl.BlockSpec((1, NHID), lambda i: (0, 0)),
            pl.BlockSpec((1, NHID), lambda i: (0, 0)),
            pl.BlockSpec((1, NHID), lambda i: (0, 0)),
            pl.BlockSpec((NHID, NCLASS), lambda i: (0, 0)),
            pl.BlockSpec((1, NCLASS), lambda i: (0, 0)),
        ],
        out_specs=[
            pl.BlockSpec((BM, NCLASS), lambda i: (i, 0)),
            pl.BlockSpec((BM, NHID), lambda i: (i, 0)),
        ],
        out_shape=[
            jax.ShapeDtypeStruct((N, NCLASS), f32),
            jax.ShapeDtypeStruct((N, NHID), f32),
        ],
        compiler_params=pltpu.CompilerParams(
            dimension_semantics=("parallel",)),
    )(adj, h1, h0, M2, B2, Wd0, bd0, Wd1, bd1,
      ln1_g.reshape(1, NHID), ln1_b.reshape(1, NHID),
      ln2_g.reshape(1, NHID), ln2_b.reshape(1, NHID),
      out_W, out_b.reshape(1, NCLASS))

    return (logp, cross)


# speedup vs baseline: 1.1894x; 1.0740x over previous
import math

import jax
import jax.numpy as jnp
from jax.experimental import pallas as pl
from jax.experimental.pallas import tpu as pltpu

N = 10000
NFEAT = 128
NHID = 128
NCLASS = 40
LAMDA = 0.5
ALPHA = 0.1

BM = 400
NB = N // BM


def _ln(z, g, b):
    m = jnp.mean(z, axis=-1, keepdims=True)
    c = z - m
    v = jnp.mean(c * c, axis=-1, keepdims=True)
    return g * c * jax.lax.rsqrt(v + 1e-6) + b


def _fused_body(adj_ref, x_ref, fcw_ref, fcb_ref, m1_ref, b1_ref,
                m2_ref, b2_ref, wd0_ref, bd0_ref, wd1_ref, bd1_ref,
                ln1g_ref, ln1b_ref, ln2g_ref, ln2b_ref, ow_ref, ob_ref,
                logp_ref, cross_ref, h0_ref, h1_ref):
    i = pl.program_id(0)

    @pl.when(i == 0)
    def _input_proj():
        h0_ref[...] = jax.nn.relu(
            jnp.dot(x_ref[...], fcw_ref[...],
                    preferred_element_type=jnp.float32) + fcb_ref[...])

    @pl.when((i >= 1) & (i <= NB))
    def _layer1():
        b = i - 1
        hi = jnp.dot(adj_ref[...], h0_ref[...],
                     preferred_element_type=jnp.float32)
        h0_rows = h0_ref[pl.ds(b * BM, BM), :]
        h1_ref[pl.ds(b * BM, BM), :] = jax.nn.relu(
            jnp.dot(hi, m1_ref[...], preferred_element_type=jnp.float32)
            + jnp.dot(h0_rows, b1_ref[...],
                      preferred_element_type=jnp.float32))

    @pl.when(i > NB)
    def _layer2_tail():
        b = i - (NB + 1)
        hi = jnp.dot(adj_ref[...], h1_ref[...],
                     preferred_element_type=jnp.float32)
        rows = pl.ds(b * BM, BM)
        h1_rows = h1_ref[rows, :]
        h0_rows = h0_ref[rows, :]
        h2 = jax.nn.relu(
            jnp.dot(hi, m2_ref[...], preferred_element_type=jnp.float32)
            + jnp.dot(h0_rows, b2_ref[...],
                      preferred_element_type=jnp.float32)
            + h1_rows)
        dy = jax.nn.relu(
            jnp.dot(h0_rows, wd0_ref[...],
                    preferred_element_type=jnp.float32) + bd0_ref[...])
        dy = jax.nn.relu(
            jnp.dot(dy, wd1_ref[...],
                    preferred_element_type=jnp.float32)
            + bd1_ref[...]) + 0.1 * h0_rows
        cross = (_ln(h2, ln1g_ref[...], ln1b_ref[...])
                 + _ln(dy, ln2g_ref[...], ln2b_ref[...]))
        cross_ref[...] = cross
        logits = jnp.dot(cross, ow_ref[...],
                         preferred_element_type=jnp.float32) + ob_ref[...]
        mx = jnp.max(logits, axis=-1, keepdims=True)
        s = logits - mx
        logp_ref[...] = s - jnp.log(jnp.sum(jnp.exp(s), axis=-1,
                                            keepdims=True))


def kernel(x, adj, fc_W, fc_b, conv_W, dyn_W, dyn_b,
           ln1_g, ln1_b, ln2_g, ln2_b, out_W, out_b):
    f32 = jnp.float32
    eye = jnp.eye(NHID, dtype=f32)
    th1 = math.log(LAMDA / 1.0 + 1.0)
    th2 = math.log(LAMDA / 2.0 + 1.0)
    M1 = th1 * conv_W[0, :NHID] + (1.0 - th1) * (1.0 - ALPHA) * eye
    B1 = th1 * conv_W[0, NHID:] + ((1.0 - th1) * ALPHA + 1.0) * eye
    M2 = th2 * conv_W[1, :NHID] + (1.0 - th2) * (1.0 - ALPHA) * eye
    B2 = th2 * conv_W[1, NHID:] + (1.0 - th2) * ALPHA * eye
    Wd0 = dyn_W[0, :NHID] - dyn_W[0, NHID:]
    Wd1 = dyn_W[1, :NHID] - dyn_W[1, NHID:]

    def adj_idx(i):
        return (jnp.where(i == 0, 0, (i - 1) % NB), 0)

    def out_idx(i):
        return (jnp.maximum(i - (NB + 1), 0), 0)

    def const_idx(i):
        return (0, 0)

    logp, cross = pl.pallas_call(
        _fused_body,
        grid=(1 + 2 * NB,),
        in_specs=[
            pl.BlockSpec((BM, N), adj_idx),
            pl.BlockSpec((N, NFEAT), const_idx),
            pl.BlockSpec((NFEAT, NHID), const_idx),
            pl.BlockSpec((1, NHID), const_idx),
            pl.BlockSpec((NHID, NHID), const_idx),
            pl.BlockSpec((NHID, NHID), const_idx),
            pl.BlockSpec((NHID, NHID), const_idx),
            pl.BlockSpec((NHID, NHID), const_idx),
            pl.BlockSpec((NHID, NHID), const_idx),
            pl.BlockSpec((1, NHID), const_idx),
            pl.BlockSpec((NHID, NHID), const_idx),
            pl.BlockSpec((1, NHID), const_idx),
            pl.BlockSpec((1, NHID), const_idx),
            pl.BlockSpec((1, NHID), const_idx),
            pl.BlockSpec((1, NHID), const_idx),
            pl.BlockSpec((1, NHID), const_idx),
            pl.BlockSpec((NHID, NCLASS), const_idx),
            pl.BlockSpec((1, NCLASS), const_idx),
        ],
        out_specs=[
            pl.BlockSpec((BM, NCLASS), out_idx),
            pl.BlockSpec((BM, NHID), out_idx),
        ],
        out_shape=[
            jax.ShapeDtypeStruct((N, NCLASS), f32),
            jax.ShapeDtypeStruct((N, NHID), f32),
        ],
        scratch_shapes=[
            pltpu.VMEM((N, NHID), f32),
            pltpu.VMEM((N, NHID), f32),
        ],
        compiler_params=pltpu.CompilerParams(
            dimension_semantics=("arbitrary",)),
    )(adj, x, fc_W, fc_b.reshape(1, NHID), M1, B1, M2, B2,
      Wd0, dyn_b[0].reshape(1, NHID), Wd1, dyn_b[1].reshape(1, NHID),
      ln1_g.reshape(1, NHID), ln1_b.reshape(1, NHID),
      ln2_g.reshape(1, NHID), ln2_b.reshape(1, NHID),
      out_W, out_b.reshape(1, NCLASS))

    return (logp, cross)
